# baseline (speedup 1.0000x reference)
import functools

import jax
import jax.numpy as jnp
from jax import lax
from jax.experimental import pallas as pl
from jax.experimental.pallas import tpu as pltpu
from jax.experimental.pallas import tpu_sc as plsc

_NPOINT = 2048
_L = 16
_NC = 2
_NS = 16
_SHARD = 4


def _fps_sc(x, y, z, npoint):
    B, N = x.shape
    assert B * _SHARD == _NC * _NS
    Nl = N // _SHARD
    nchunks = Nl // _L
    mesh = plsc.VectorSubcoreMesh(
        core_axis_name="c", subcore_axis_name="s", num_cores=_NC, num_subcores=_NS
    )

    @functools.partial(
        pl.kernel,
        out_type=[jax.ShapeDtypeStruct((B, npoint), jnp.float32)] * 3,
        mesh=mesh,
        compiler_params=pltpu.CompilerParams(needs_layout_passes=False, use_tc_tiling_on_sc=False),
        scratch_types=[
            pltpu.VMEM((Nl,), jnp.float32),
            pltpu.VMEM((Nl,), jnp.float32),
            pltpu.VMEM((Nl,), jnp.float32),
            pltpu.VMEM((Nl,), jnp.float32),
            pltpu.VMEM((npoint,), jnp.float32),
            pltpu.VMEM((npoint,), jnp.float32),
            pltpu.VMEM((npoint,), jnp.float32),
            pltpu.VMEM((_L,), jnp.float32),
            pltpu.VMEM((_SHARD, _L), jnp.float32),
            pltpu.VMEM_SHARED((2, _NS, _L), jnp.float32),
        ],
    )
    def k(xh, yh, zh, oxh, oyh, ozh, xv, yv, zv, dv, oxv, oyv, ozv, rowv, candv, spm):
        cid = lax.axis_index("c")
        sid = lax.axis_index("s")
        b = cid * (_NS // _SHARD) + sid // _SHARD
        mem = sid % _SHARD
        base = mem * Nl
        g0 = (sid // _SHARD) * _SHARD

        pltpu.sync_copy(xh.at[b, pl.ds(base, Nl)], xv)
        pltpu.sync_copy(yh.at[b, pl.ds(base, Nl)], yv)
        pltpu.sync_copy(zh.at[b, pl.ds(base, Nl)], zv)

        big = jnp.full((_L,), 1e10, jnp.float32)

        def initc(kk, carry):
            dv[pl.ds(kk * _L, _L)] = big
            return carry

        lax.fori_loop(0, nchunks, initc, 0, unroll=8)

        lanes = lax.iota(jnp.int32, _L)
        lane0 = lanes == 0

        pltpu.sync_copy(xh.at[b, pl.ds(0, _L)], rowv)
        cx0 = rowv[...][0]
        pltpu.sync_copy(yh.at[b, pl.ds(0, _L)], rowv)
        cy0 = rowv[...][0]
        pltpu.sync_copy(zh.at[b, pl.ds(0, _L)], rowv)
        cz0 = rowv[...][0]

        def step(i, carry):
            cx, cy, cz = carry
            iv = jnp.broadcast_to(i, (_L,))
            plsc.store_scatter(oxv, [iv], jnp.broadcast_to(cx, (_L,)), mask=lane0)
            plsc.store_scatter(oyv, [iv], jnp.broadcast_to(cy, (_L,)), mask=lane0)
            plsc.store_scatter(ozv, [iv], jnp.broadcast_to(cz, (_L,)), mask=lane0)

            def chunk(kk, c2):
                bval, bidx = c2
                off = pl.multiple_of(kk * _L, _L)
                dx = xv[pl.ds(off, _L)] - cx
                dy = yv[pl.ds(off, _L)] - cy
                dz = zv[pl.ds(off, _L)] - cz
                sxy = dx * dx + dy * dy
                d = sxy + dz * dz
                nd = jnp.minimum(dv[pl.ds(off, _L)], d)
                dv[pl.ds(off, _L)] = nd
                m = nd > bval
                bval = jnp.where(m, nd, bval)
                bidx = jnp.where(m, lanes + off, bidx)
                return bval, bidx

            bval0 = jnp.full((_L,), -1.0, jnp.float32)
            bidx0 = jnp.zeros((_L,), jnp.int32)
            bval, bidx = plsc.parallel_loop(
                0, nchunks, carry=(bval0, bidx0), unroll=16
            )(chunk)
            mval = jnp.max(bval)
            cand = jnp.where(bval == mval, bidx, jnp.int32(Nl))
            lidx = jnp.min(cand)
            lv = jnp.broadcast_to(lidx, (_L,))
            cxl = plsc.load_gather(xv, [lv])[0]
            cyl = plsc.load_gather(yv, [lv])[0]
            czl = plsc.load_gather(zv, [lv])[0]

            row = jnp.broadcast_to(czl, (_L,))
            row = jnp.where(lanes == 0, jnp.broadcast_to(mval, (_L,)), row)
            row = jnp.where(lanes == 1, jnp.broadcast_to(cxl, (_L,)), row)
            row = jnp.where(lanes == 2, jnp.broadcast_to(cyl, (_L,)), row)
            rowv[...] = row
            p = lax.rem(i, 2)
            pltpu.sync_copy(rowv, spm.at[p, sid])
            plsc.subcore_barrier()
            pltpu.sync_copy(spm.at[p, pl.ds(g0, _SHARD), :], candv)

            r0 = candv[0, :]
            bv, bx, by, bz = r0[0], r0[1], r0[2], r0[3]
            for r in range(1, _SHARD):
                rr = candv[r, :]
                take = rr[0] > bv
                bv = jnp.where(take, rr[0], bv)
                bx = jnp.where(take, rr[1], bx)
                by = jnp.where(take, rr[2], by)
                bz = jnp.where(take, rr[3], bz)
            return bx, by, bz

        lax.fori_loop(0, npoint, step, (cx0, cy0, cz0))

        @pl.when(mem == 0)
        def _():
            pltpu.sync_copy(oxv, oxh.at[b])
            pltpu.sync_copy(oyv, oyh.at[b])
            pltpu.sync_copy(ozv, ozh.at[b])

    return k(x, y, z)


def kernel(points_xyz, points_xyz_t, features_with_xyz):
    x = points_xyz_t[:, 0, :]
    y = points_xyz_t[:, 1, :]
    z = points_xyz_t[:, 2, :]
    ox, oy, oz = _fps_sc(x, y, z, _NPOINT)
    return jnp.stack([ox, oy, oz], axis=1)


# speedup vs baseline: 1.0892x; 1.0892x over previous
import functools

import jax
import jax.numpy as jnp
from jax import lax
from jax.experimental import pallas as pl
from jax.experimental.pallas import tpu as pltpu
from jax.experimental.pallas import tpu_sc as plsc

_NPOINT = 2048
_L = 16
_NC = 2
_NS = 16
_SHARD = 4


def _fps_sc(x, y, z, npoint):
    B, N = x.shape
    assert B * _SHARD == _NC * _NS
    Nl = N // _SHARD
    nchunks = Nl // _L
    mesh = plsc.VectorSubcoreMesh(
        core_axis_name="c", subcore_axis_name="s", num_cores=_NC, num_subcores=_NS
    )

    @functools.partial(
        pl.kernel,
        out_type=[jax.ShapeDtypeStruct((B, npoint), jnp.float32)] * 3,
        mesh=mesh,
        compiler_params=pltpu.CompilerParams(needs_layout_passes=False, use_tc_tiling_on_sc=False),
        scratch_types=[
            pltpu.VMEM((Nl,), jnp.float32),
            pltpu.VMEM((Nl,), jnp.float32),
            pltpu.VMEM((Nl,), jnp.float32),
            pltpu.VMEM((Nl,), jnp.float32),
            pltpu.VMEM((npoint,), jnp.float32),
            pltpu.VMEM((npoint,), jnp.float32),
            pltpu.VMEM((npoint,), jnp.float32),
            pltpu.VMEM((_L,), jnp.float32),
            pltpu.VMEM((_SHARD, _L), jnp.float32),
            pltpu.VMEM_SHARED((2, _NS, _L), jnp.float32),
        ],
    )
    def k(xh, yh, zh, oxh, oyh, ozh, xv, yv, zv, dv, oxv, oyv, ozv, rowv, candv, spm):
        cid = lax.axis_index("c")
        sid = lax.axis_index("s")
        b = cid * (_NS // _SHARD) + sid // _SHARD
        mem = sid % _SHARD
        base = mem * Nl
        g0 = (sid // _SHARD) * _SHARD

        pltpu.sync_copy(xh.at[b, pl.ds(base, Nl)], xv)
        pltpu.sync_copy(yh.at[b, pl.ds(base, Nl)], yv)
        pltpu.sync_copy(zh.at[b, pl.ds(base, Nl)], zv)

        big = jnp.full((_L,), 1e10, jnp.float32)

        def initc(kk, carry):
            dv[pl.ds(kk * _L, _L)] = big
            return carry

        lax.fori_loop(0, nchunks, initc, 0, unroll=8)

        lanes = lax.iota(jnp.int32, _L)
        lane0 = lanes == 0

        pltpu.sync_copy(xh.at[b, pl.ds(0, _L)], rowv)
        cx0 = rowv[...][0]
        pltpu.sync_copy(yh.at[b, pl.ds(0, _L)], rowv)
        cy0 = rowv[...][0]
        pltpu.sync_copy(zh.at[b, pl.ds(0, _L)], rowv)
        cz0 = rowv[...][0]

        def step(i, carry):
            cx, cy, cz = carry
            iv = jnp.broadcast_to(i, (_L,))
            plsc.store_scatter(oxv, [iv], jnp.broadcast_to(cx, (_L,)), mask=lane0)
            plsc.store_scatter(oyv, [iv], jnp.broadcast_to(cy, (_L,)), mask=lane0)
            plsc.store_scatter(ozv, [iv], jnp.broadcast_to(cz, (_L,)), mask=lane0)

            def chunk(kk, c2):
                bval, bidx = c2
                off = pl.multiple_of(kk * _L, _L)
                dx = xv[pl.ds(off, _L)] - cx
                dy = yv[pl.ds(off, _L)] - cy
                dz = zv[pl.ds(off, _L)] - cz
                sxy = dx * dx + dy * dy
                d = sxy + dz * dz
                nd = jnp.minimum(dv[pl.ds(off, _L)], d)
                dv[pl.ds(off, _L)] = nd
                m = nd > bval
                bval = jnp.where(m, nd, bval)
                bidx = jnp.where(m, lanes + off, bidx)
                return bval, bidx

            bval0 = jnp.full((_L,), -1.0, jnp.float32)
            bidx0 = jnp.zeros((_L,), jnp.int32)
            bval, bidx = plsc.parallel_loop(
                0, nchunks, carry=(bval0, bidx0), unroll=4
            )(chunk)
            mval = jnp.max(bval)
            cand = jnp.where(bval == mval, bidx, jnp.int32(Nl))
            lidx = jnp.min(cand)
            lv = jnp.broadcast_to(lidx, (_L,))
            cxl = plsc.load_gather(xv, [lv])[0]
            cyl = plsc.load_gather(yv, [lv])[0]
            czl = plsc.load_gather(zv, [lv])[0]

            row = jnp.broadcast_to(czl, (_L,))
            row = jnp.where(lanes == 0, jnp.broadcast_to(mval, (_L,)), row)
            row = jnp.where(lanes == 1, jnp.broadcast_to(cxl, (_L,)), row)
            row = jnp.where(lanes == 2, jnp.broadcast_to(cyl, (_L,)), row)
            rowv[...] = row
            p = lax.rem(i, 2)
            pltpu.sync_copy(rowv, spm.at[p, sid])
            plsc.subcore_barrier()
            pltpu.sync_copy(spm.at[p, pl.ds(g0, _SHARD), :], candv)

            r0 = candv[0, :]
            bv, bx, by, bz = r0[0], r0[1], r0[2], r0[3]
            for r in range(1, _SHARD):
                rr = candv[r, :]
                take = rr[0] > bv
                bv = jnp.where(take, rr[0], bv)
                bx = jnp.where(take, rr[1], bx)
                by = jnp.where(take, rr[2], by)
                bz = jnp.where(take, rr[3], bz)
            return bx, by, bz

        lax.fori_loop(0, npoint, step, (cx0, cy0, cz0))

        @pl.when(mem == 0)
        def _():
            pltpu.sync_copy(oxv, oxh.at[b])
            pltpu.sync_copy(oyv, oyh.at[b])
            pltpu.sync_copy(ozv, ozh.at[b])

    return k(x, y, z)


def kernel(points_xyz, points_xyz_t, features_with_xyz):
    x = points_xyz_t[:, 0, :]
    y = points_xyz_t[:, 1, :]
    z = points_xyz_t[:, 2, :]
    ox, oy, oz = _fps_sc(x, y, z, _NPOINT)
    return jnp.stack([ox, oy, oz], axis=1)
